# baseline (speedup 1.0000x reference)
import functools

import jax
import jax.numpy as jnp
from jax.experimental import pallas as pl
from jax.experimental.pallas import tpu as pltpu

_TOP_K = 8


def _router_body(x_ref, wt_ref, probs_ref, w_ref, i_ref, acc_ref, *, nk):
    part = jnp.dot(x_ref[...], wt_ref[...], preferred_element_type=jnp.float32)
    k = pl.program_id(1)

    @pl.when(k == 0)
    def _():
        acc_ref[...] = part

    @pl.when((k > 0) & (k < nk - 1))
    def _():
        acc_ref[...] += part

    @pl.when(k == nk - 1)
    def _():
        _finish(part + acc_ref[...] if nk > 1 else part, probs_ref, w_ref, i_ref)


def _finish(logits, probs_ref, w_ref, i_ref):
    m = jnp.max(logits, axis=-1, keepdims=True)
    e = jnp.exp(logits - m)
    s = jnp.sum(e, axis=-1, keepdims=True)
    probs = e / s
    probs_ref[...] = probs

    n_exp = probs.shape[-1]
    lane = jax.lax.broadcasted_iota(jnp.int32, probs.shape, dimension=1)
    work = probs
    ws = []
    idxs = []
    for _ in range(_TOP_K):
        mx = jnp.max(work, axis=-1, keepdims=True)
        is_max = work == mx
        cand = jnp.where(is_max, lane, n_exp)
        sel = jnp.min(cand, axis=-1, keepdims=True)
        ws.append(mx)
        idxs.append(sel)
        work = jnp.where(lane == sel, -1.0, work)
    w = jnp.concatenate(ws, axis=1)
    idx = jnp.concatenate(idxs, axis=1)
    w = w / jnp.sum(w, axis=-1, keepdims=True)
    w_ref[...] = w
    i_ref[...] = idx


def kernel(x, W):
    b, s, h = x.shape
    n_exp = W.shape[0]
    n = b * s
    xf = x.reshape(n, h)
    wt = W.T

    tile = 2048
    while n % tile:
        tile //= 2
    nk = 2
    while h % nk:
        nk //= 2
    hc = h // nk
    grid = (n // tile, nk)

    probs, w, idx = pl.pallas_call(
        functools.partial(_router_body, nk=nk),
        grid=grid,
        in_specs=[
            pl.BlockSpec((tile, hc), lambda i, k: (i, k)),
            pl.BlockSpec((hc, n_exp), lambda i, k: (k, 0)),
        ],
        out_specs=[
            pl.BlockSpec((tile, n_exp), lambda i, k: (i, 0)),
            pl.BlockSpec((tile, _TOP_K), lambda i, k: (i, 0)),
            pl.BlockSpec((tile, _TOP_K), lambda i, k: (i, 0)),
        ],
        out_shape=[
            jax.ShapeDtypeStruct((n, n_exp), jnp.float32),
            jax.ShapeDtypeStruct((n, _TOP_K), jnp.float32),
            jax.ShapeDtypeStruct((n, _TOP_K), jnp.int32),
        ],
        scratch_shapes=[pltpu.VMEM((tile, n_exp), jnp.float32)],
    )(xf, wt)

    return (
        w.reshape(b, s, _TOP_K),
        idx.reshape(b, s, _TOP_K),
        probs.reshape(b, s, n_exp),
    )


# speedup vs baseline: 1.2640x; 1.2640x over previous
import functools

import jax
import jax.numpy as jnp
from jax.experimental import pallas as pl
from jax.experimental.pallas import tpu as pltpu

_TOP_K = 8


def _finish(logits, probs_ref, w_ref, i_ref, row0):
    sub = logits.shape[0]
    m = jnp.max(logits, axis=-1, keepdims=True)
    e = jnp.exp(logits - m)
    s = jnp.sum(e, axis=-1, keepdims=True)
    probs = e / s
    probs_ref[row0 : row0 + sub, :] = probs

    n_exp = probs.shape[-1]
    lane = jax.lax.broadcasted_iota(jnp.int32, probs.shape, dimension=1)
    work = probs
    ws = []
    idxs = []
    for _ in range(_TOP_K):
        mx = jnp.max(work, axis=-1, keepdims=True)
        is_max = work == mx
        cand = jnp.where(is_max, lane, n_exp)
        sel = jnp.min(cand, axis=-1, keepdims=True)
        ws.append(mx)
        idxs.append(sel)
        work = jnp.where(lane == sel, -1.0, work)
    w = jnp.concatenate(ws, axis=1)
    idx = jnp.concatenate(idxs, axis=1)
    w = w / jnp.sum(w, axis=-1, keepdims=True)
    w_ref[row0 : row0 + sub, :] = w
    i_ref[row0 : row0 + sub, :] = idx


def _router_body(*refs, ns):
    x_refs = refs[:ns]
    wt_ref = refs[ns]
    probs_ref, w_ref, i_ref = refs[ns + 1 : ns + 4]
    for s in range(ns):
        logits = jnp.dot(
            x_refs[s][...], wt_ref[...], preferred_element_type=jnp.float32
        )
        _finish(logits, probs_ref, w_ref, i_ref, s * x_refs[s].shape[0])


def kernel(x, W):
    b, s, h = x.shape
    n_exp = W.shape[0]
    n = b * s
    xf = x.reshape(n, h)
    wt = W.T

    tile = 1024
    while n % tile:
        tile //= 2
    ns = 2
    while tile % ns:
        ns //= 2
    sub = tile // ns
    grid = (n // tile,)

    def xmap(s):
        return lambda i: (ns * i + s, 0)

    in_specs = [pl.BlockSpec((sub, h), xmap(s)) for s in range(ns)]
    in_specs.append(pl.BlockSpec((h, n_exp), lambda i: (0, 0)))
    out_specs = [
        pl.BlockSpec((tile, n_exp), lambda i: (i, 0)),
        pl.BlockSpec((tile, _TOP_K), lambda i: (i, 0)),
        pl.BlockSpec((tile, _TOP_K), lambda i: (i, 0)),
    ]
    out_shape = [
        jax.ShapeDtypeStruct((n, n_exp), jnp.float32),
        jax.ShapeDtypeStruct((n, _TOP_K), jnp.float32),
        jax.ShapeDtypeStruct((n, _TOP_K), jnp.int32),
    ]

    probs, w, idx = pl.pallas_call(
        functools.partial(_router_body, ns=ns),
        grid=grid,
        in_specs=in_specs,
        out_specs=out_specs,
        out_shape=out_shape,
    )(*([xf] * ns), wt)

    return (
        w.reshape(b, s, _TOP_K),
        idx.reshape(b, s, _TOP_K),
        probs.reshape(b, s, n_exp),
    )
